# baseline (speedup 1.0000x reference)
import functools

import jax
import jax.numpy as jnp
from jax import lax
from jax.experimental import pallas as pl
from jax.experimental.pallas import tpu as pltpu
from jax.experimental.pallas import tpu_sc as plsc

_BATCH = 16384
_FIELDS = 26
_V = 1000000
_NC = 2
_NS = 16
_NW = _NC * _NS
_COLS = _BATCH // _NW
_HCOLS = _COLS // 2
_HE = _FIELDS * _HCOLS
_VPR = _HCOLS // 16

_CHUNK = 62504
_TAIL = _V - 15 * _CHUNK
_SCHUNK = 4096
_NFULL = _CHUNK // _SCHUNK
_BTAIL = _CHUNK - _NFULL * _SCHUNK
_TTAIL = _TAIL - _NFULL * _SCHUNK
_RING = 4

_mesh = plsc.VectorSubcoreMesh(core_axis_name="c", subcore_axis_name="s")


@functools.partial(
    pl.kernel,
    mesh=_mesh,
    out_type=jax.ShapeDtypeStruct((_FIELDS, _BATCH), jnp.float32),
    scratch_types=[
        pltpu.VMEM((_FIELDS, _HCOLS), jnp.int32),
        pltpu.VMEM((_HE,), jnp.int32),
        pltpu.VMEM((_HE,), jnp.int32),
        pltpu.VMEM((_HE,), jnp.float32),
        pltpu.VMEM((_HE,), jnp.float32),
        pltpu.VMEM((_FIELDS, _HCOLS), jnp.float32),
        pltpu.VMEM((_SCHUNK,), jnp.float32),
        pltpu.VMEM((_SCHUNK,), jnp.float32),
        pltpu.VMEM((_SCHUNK,), jnp.float32),
        pltpu.VMEM((_SCHUNK,), jnp.float32),
        pltpu.VMEM_SHARED((_V,), jnp.float32),
        pltpu.SemaphoreType.DMA,
        pltpu.SemaphoreType.DMA,
        pltpu.SemaphoreType.DMA,
        pltpu.SemaphoreType.DMA,
    ],
)
def _gather_sc(idx_hbm, table_hbm, out_hbm, idx2_v, idx_a, idx_b, vals_a,
               vals_b, vals2_v, stage_0, stage_1, stage_2, stage_3, table_sp,
               sem1, sem2, gsem_a, gsem_b):
    s = lax.axis_index("s")
    wid = s * _NC + lax.axis_index("c")
    col0 = wid * _COLS
    stage = (stage_0, stage_1, stage_2, stage_3)
    idx_flat = (idx_a, idx_b)
    vals = (vals_a, vals_b)
    gsem = (gsem_a, gsem_b)

    def _flatten_half(hb):
        pltpu.sync_copy(idx_hbm.at[:, pl.ds(col0 + hb * _HCOLS, _HCOLS)],
                        idx2_v)
        dst = idx_flat[hb]

        def _row(f, _):
            for j in range(_VPR):
                dst[pl.ds(f * _HCOLS + j * 16, 16)] = \
                    idx2_v[f, pl.ds(j * 16, 16)]
            return _

        lax.fori_loop(0, _FIELDS, _row, None)

    def _stage_pipeline(pieces):
        np_ = len(pieces)

        def _fire_h1(j):
            off, sz = pieces[j]
            return pltpu.async_copy(table_hbm.at[pl.ds(off, sz)],
                                    stage[j % _RING].at[pl.ds(0, sz)], sem1)

        lead = _RING // 2
        h1 = [_fire_h1(j) for j in range(min(lead, np_))]
        h1 += [None] * (np_ - len(h1))
        h2 = [None] * np_
        done = 0
        for j, (off, sz) in enumerate(pieces):
            h1[j].wait()
            h2[j] = pltpu.async_copy(stage[j % _RING].at[pl.ds(0, sz)],
                                     table_sp.at[pl.ds(off, sz)], sem2)
            if done < 2 and j in (1, 5):
                _flatten_half(done)
                done += 1
            nxt = j + lead
            if nxt < np_:
                if nxt - _RING >= 0:
                    h2[nxt - _RING].wait()
                    h2[nxt - _RING] = False
                h1[nxt] = _fire_h1(nxt)
        for j in range(np_):
            if h2[j] not in (None, False):
                h2[j].wait()
        while done < 2:
            _flatten_half(done)
            done += 1

    @pl.when(s < _NS - 1)
    def _stage_body():
      with jax.named_scope("stage"):
        base = pl.multiple_of(s * _CHUNK, 8)
        pieces = [(pl.multiple_of(base + j * _SCHUNK, 8), _SCHUNK)
                  for j in range(_NFULL)]
        pieces.append((pl.multiple_of(base + _NFULL * _SCHUNK, 8), _BTAIL))
        _stage_pipeline(pieces)

    @pl.when(s == _NS - 1)
    def _stage_tail():
        pieces = [(15 * _CHUNK + j * _SCHUNK, _SCHUNK)
                  for j in range(_NFULL)]
        pieces.append((15 * _CHUNK + _NFULL * _SCHUNK, _TTAIL))
        _stage_pipeline(pieces)

    with jax.named_scope("barrier"):
        plsc.subcore_barrier()

    g0 = pltpu.async_copy(table_sp.at[idx_a], vals_a, gsem_a)
    g1 = pltpu.async_copy(table_sp.at[idx_b], vals_b, gsem_b)

    for hb, g in ((0, g0), (1, g1)):
        with jax.named_scope(f"gwait{hb}"):
            g.wait()
        vb = vals[hb]

        def _row(f, _):
            for j in range(_VPR):
                vals2_v[f, pl.ds(j * 16, 16)] = \
                    vb[pl.ds(f * _HCOLS + j * 16, 16)]
            return _

        lax.fori_loop(0, _FIELDS, _row, None)
        pltpu.sync_copy(vals2_v,
                        out_hbm.at[:, pl.ds(col0 + hb * _HCOLS, _HCOLS)])


def kernel(ind, scales):
    if ind.dtype != jnp.int32:
        ind = ind.astype(jnp.int32)
    return _gather_sc(ind.T, scales).T


# speedup vs baseline: 1.0542x; 1.0542x over previous
import functools

import jax
import jax.numpy as jnp
from jax import lax
from jax.experimental import pallas as pl
from jax.experimental.pallas import tpu as pltpu
from jax.experimental.pallas import tpu_sc as plsc

_BATCH = 16384
_FIELDS = 26
_V = 1000000
_NC = 2
_NS = 16
_NW = _NC * _NS
_COLS = _BATCH // _NW
_HCOLS = _COLS // 2
_HE = _FIELDS * _HCOLS
_VPR = _HCOLS // 16

_CHUNK = 62504
_TAIL = _V - 15 * _CHUNK
_SCHUNK = 4096
_NFULL = _CHUNK // _SCHUNK
_BTAIL = _CHUNK - _NFULL * _SCHUNK
_TTAIL = _TAIL - _NFULL * _SCHUNK
_RING = 6

_mesh = plsc.VectorSubcoreMesh(core_axis_name="c", subcore_axis_name="s")


@functools.partial(
    pl.kernel,
    mesh=_mesh,
    out_type=jax.ShapeDtypeStruct((_FIELDS, _BATCH), jnp.float32),
    scratch_types=[
        pltpu.VMEM((_FIELDS, _HCOLS), jnp.int32),
        pltpu.VMEM((_HE,), jnp.int32),
        pltpu.VMEM((_HE,), jnp.int32),
        pltpu.VMEM((_HE,), jnp.float32),
        pltpu.VMEM((_HE,), jnp.float32),
        pltpu.VMEM((_FIELDS, _HCOLS), jnp.float32),
        pltpu.VMEM((_SCHUNK,), jnp.float32),
        pltpu.VMEM((_SCHUNK,), jnp.float32),
        pltpu.VMEM((_SCHUNK,), jnp.float32),
        pltpu.VMEM((_SCHUNK,), jnp.float32),
        pltpu.VMEM((_SCHUNK,), jnp.float32),
        pltpu.VMEM((_SCHUNK,), jnp.float32),
        pltpu.VMEM_SHARED((_V,), jnp.float32),
        pltpu.SemaphoreType.DMA,
        pltpu.SemaphoreType.DMA,
        pltpu.SemaphoreType.DMA,
        pltpu.SemaphoreType.DMA,
    ],
)
def _gather_sc(idx_hbm, table_hbm, out_hbm, idx2_v, idx_a, idx_b, vals_a,
               vals_b, vals2_v, stage_0, stage_1, stage_2, stage_3, stage_4,
               stage_5, table_sp,
               sem1, sem2, gsem_a, gsem_b):
    s = lax.axis_index("s")
    wid = s * _NC + lax.axis_index("c")
    col0 = wid * _COLS
    stage = (stage_0, stage_1, stage_2, stage_3, stage_4, stage_5)
    idx_flat = (idx_a, idx_b)
    vals = (vals_a, vals_b)
    gsem = (gsem_a, gsem_b)

    def _flatten_half(hb):
        pltpu.sync_copy(idx_hbm.at[:, pl.ds(col0 + hb * _HCOLS, _HCOLS)],
                        idx2_v)
        dst = idx_flat[hb]

        def _row(f, _):
            for j in range(_VPR):
                dst[pl.ds(f * _HCOLS + j * 16, 16)] = \
                    idx2_v[f, pl.ds(j * 16, 16)]
            return _

        lax.fori_loop(0, _FIELDS, _row, None)

    def _stage_pipeline(pieces):
        np_ = len(pieces)

        def _fire_h1(j):
            off, sz = pieces[j]
            return pltpu.async_copy(table_hbm.at[pl.ds(off, sz)],
                                    stage[j % _RING].at[pl.ds(0, sz)], sem1)

        lead = _RING // 2
        h1 = [_fire_h1(j) for j in range(min(lead, np_))]
        h1 += [None] * (np_ - len(h1))
        h2 = [None] * np_
        done = 0
        for j, (off, sz) in enumerate(pieces):
            h1[j].wait()
            h2[j] = pltpu.async_copy(stage[j % _RING].at[pl.ds(0, sz)],
                                     table_sp.at[pl.ds(off, sz)], sem2)
            if done < 2 and j in (1, 5):
                _flatten_half(done)
                done += 1
            nxt = j + lead
            if nxt < np_:
                if nxt - _RING >= 0:
                    h2[nxt - _RING].wait()
                    h2[nxt - _RING] = False
                h1[nxt] = _fire_h1(nxt)
        for j in range(np_):
            if h2[j] not in (None, False):
                h2[j].wait()
        while done < 2:
            _flatten_half(done)
            done += 1

    @pl.when(s < _NS - 1)
    def _stage_body():
      with jax.named_scope("stage"):
        base = pl.multiple_of(s * _CHUNK, 8)
        pieces = [(pl.multiple_of(base + j * _SCHUNK, 8), _SCHUNK)
                  for j in range(_NFULL)]
        pieces.append((pl.multiple_of(base + _NFULL * _SCHUNK, 8), _BTAIL))
        _stage_pipeline(pieces)

    @pl.when(s == _NS - 1)
    def _stage_tail():
        pieces = [(15 * _CHUNK + j * _SCHUNK, _SCHUNK)
                  for j in range(_NFULL)]
        pieces.append((15 * _CHUNK + _NFULL * _SCHUNK, _TTAIL))
        _stage_pipeline(pieces)

    with jax.named_scope("barrier"):
        plsc.subcore_barrier()

    g0 = pltpu.async_copy(table_sp.at[idx_a], vals_a, gsem_a)
    g1 = pltpu.async_copy(table_sp.at[idx_b], vals_b, gsem_b)

    for hb, g in ((0, g0), (1, g1)):
        with jax.named_scope(f"gwait{hb}"):
            g.wait()
        vb = vals[hb]

        def _row(f, _):
            for j in range(_VPR):
                vals2_v[f, pl.ds(j * 16, 16)] = \
                    vb[pl.ds(f * _HCOLS + j * 16, 16)]
            return _

        lax.fori_loop(0, _FIELDS, _row, None)
        pltpu.sync_copy(vals2_v,
                        out_hbm.at[:, pl.ds(col0 + hb * _HCOLS, _HCOLS)])


def kernel(ind, scales):
    if ind.dtype != jnp.int32:
        ind = ind.astype(jnp.int32)
    return _gather_sc(ind.T, scales).T


# speedup vs baseline: 1.0628x; 1.0082x over previous
import functools

import jax
import jax.numpy as jnp
from jax import lax
from jax.experimental import pallas as pl
from jax.experimental.pallas import tpu as pltpu
from jax.experimental.pallas import tpu_sc as plsc

_BATCH = 16384
_FIELDS = 26
_V = 1000000
_NC = 2
_NS = 16
_NW = _NC * _NS
_COLS = _BATCH // _NW
_HCOLS = _COLS // 2
_HE = _FIELDS * _HCOLS
_VPR = _HCOLS // 16

_CHUNK = 62504
_TAIL = _V - 15 * _CHUNK
_SCHUNK = 4096
_NFULL = _CHUNK // _SCHUNK
_BTAIL = _CHUNK - _NFULL * _SCHUNK
_TTAIL = _TAIL - _NFULL * _SCHUNK
_RING = 4

_mesh = plsc.VectorSubcoreMesh(core_axis_name="c", subcore_axis_name="s")


@functools.partial(
    pl.kernel,
    mesh=_mesh,
    out_type=jax.ShapeDtypeStruct((_FIELDS, _BATCH), jnp.float32),
    scratch_types=[
        pltpu.VMEM((_FIELDS, _HCOLS), jnp.int32),
        pltpu.VMEM((_HE,), jnp.int32),
        pltpu.VMEM((_HE,), jnp.int32),
        pltpu.VMEM((_HE,), jnp.float32),
        pltpu.VMEM((_HE,), jnp.float32),
        pltpu.VMEM((_FIELDS, _HCOLS), jnp.float32),
        pltpu.VMEM((_SCHUNK,), jnp.float32),
        pltpu.VMEM((_SCHUNK,), jnp.float32),
        pltpu.VMEM((_SCHUNK,), jnp.float32),
        pltpu.VMEM((_SCHUNK,), jnp.float32),
        pltpu.VMEM_SHARED((_V,), jnp.float32),
        pltpu.SemaphoreType.DMA,
        pltpu.SemaphoreType.DMA,
        pltpu.SemaphoreType.DMA,
        pltpu.SemaphoreType.DMA,
    ],
)
def _gather_sc(idx_hbm, table_hbm, out_hbm, idx2_v, idx_a, idx_b, vals_a,
               vals_b, vals2_v, stage_0, stage_1, stage_2, stage_3, table_sp,
               sem1, sem2, gsem_a, gsem_b):
    s = lax.axis_index("s")
    wid = s * _NC + lax.axis_index("c")
    col0 = wid * _COLS
    stage = (stage_0, stage_1, stage_2, stage_3)
    idx_flat = (idx_a, idx_b)
    vals = (vals_a, vals_b)
    gsem = (gsem_a, gsem_b)

    def _flatten_half(hb):
        pltpu.sync_copy(idx_hbm.at[:, pl.ds(col0 + hb * _HCOLS, _HCOLS)],
                        idx2_v)
        dst = idx_flat[hb]

        def _row(f, _):
            for j in range(_VPR):
                dst[pl.ds(f * _HCOLS + j * 16, 16)] = \
                    idx2_v[f, pl.ds(j * 16, 16)]
            return _

        lax.fori_loop(0, _FIELDS, _row, None)

    def _stage_pipeline(pieces):
        np_ = len(pieces)

        def _fire_h1(j):
            off, sz = pieces[j]
            return pltpu.async_copy(table_hbm.at[pl.ds(off, sz)],
                                    stage[j % _RING].at[pl.ds(0, sz)], sem1)

        h1 = [_fire_h1(j) for j in range(min(_RING, np_))]
        h2 = [None] * _RING
        done = 0
        for j, (off, sz) in enumerate(pieces):
            slot = j % _RING
            h1[slot].wait()
            h2[slot] = pltpu.async_copy(stage[slot].at[pl.ds(0, sz)],
                                        table_sp.at[pl.ds(off, sz)], sem2)
            if done < 2 and j in (1, 5):
                _flatten_half(done)
                done += 1
            if j + _RING < np_:
                h2[slot].wait()
                h1[slot] = _fire_h1(j + _RING)
                h2[slot] = None
        for slot in range(_RING):
            if h2[slot] is not None:
                h2[slot].wait()
        while done < 2:
            _flatten_half(done)
            done += 1

    @pl.when(s < _NS - 1)
    def _stage_body():
        base = pl.multiple_of(s * _CHUNK, 8)
        pieces = [(pl.multiple_of(base + j * _SCHUNK, 8), _SCHUNK)
                  for j in range(_NFULL)]
        pieces.append((pl.multiple_of(base + _NFULL * _SCHUNK, 8), _BTAIL))
        _stage_pipeline(pieces)

    @pl.when(s == _NS - 1)
    def _stage_tail():
        pieces = [(15 * _CHUNK + j * _SCHUNK, _SCHUNK)
                  for j in range(_NFULL)]
        pieces.append((15 * _CHUNK + _NFULL * _SCHUNK, _TTAIL))
        _stage_pipeline(pieces)

    plsc.subcore_barrier()

    g0 = pltpu.async_copy(table_sp.at[idx_a], vals_a, gsem_a)
    g1 = pltpu.async_copy(table_sp.at[idx_b], vals_b, gsem_b)

    for hb, g in ((0, g0), (1, g1)):
        g.wait()
        vb = vals[hb]

        def _row(f, _):
            for j in range(_VPR):
                vals2_v[f, pl.ds(j * 16, 16)] = \
                    vb[pl.ds(f * _HCOLS + j * 16, 16)]
            return _

        lax.fori_loop(0, _FIELDS, _row, None)
        pltpu.sync_copy(vals2_v,
                        out_hbm.at[:, pl.ds(col0 + hb * _HCOLS, _HCOLS)])


def kernel(ind, scales):
    if ind.dtype != jnp.int32:
        ind = ind.astype(jnp.int32)
    return _gather_sc(ind.T, scales).T


# speedup vs baseline: 1.2382x; 1.1650x over previous
import jax
import jax.numpy as jnp
from jax import lax
from jax.experimental import pallas as pl
from jax.experimental.pallas import tpu as pltpu
from jax.experimental.pallas import tpu_sc as plsc
from jax._src.pallas import mpmd

_BATCH = 16384
_FIELDS = 26
_V = 1000000
_NC = 2
_NS = 16
_NW = _NC * _NS
_COLS = _BATCH // _NW
_HCOLS = _COLS // 2
_HE = _FIELDS * _HCOLS
_VPR = _HCOLS // 16

_scs_mesh = plsc.ScalarSubcoreMesh(axis_name="c")
_vec_mesh = plsc.VectorSubcoreMesh(core_axis_name="c", subcore_axis_name="s")


def _scs_fn(idx_hbm, table_hbm, out_hbm, idx2_v, idx_a, idx_b, vals_a,
            vals_b, vals2_v, table_sp, gsem_a, gsem_b, sem_ready):
    pltpu.sync_copy(table_hbm, table_sp)
    for t in range(_NS):
        pl.semaphore_signal(sem_ready, 1, device_id={"s": t})


def _tec_fn(idx_hbm, table_hbm, out_hbm, idx2_v, idx_a, idx_b, vals_a,
            vals_b, vals2_v, table_sp, gsem_a, gsem_b, sem_ready):
    s = lax.axis_index("s")
    wid = s * _NC + lax.axis_index("c")
    col0 = wid * _COLS
    idx_flat = (idx_a, idx_b)
    vals = (vals_a, vals_b)

    for hb in range(2):
        pltpu.sync_copy(idx_hbm.at[:, pl.ds(col0 + hb * _HCOLS, _HCOLS)],
                        idx2_v)
        dst = idx_flat[hb]

        def _row(f, _):
            for j in range(_VPR):
                dst[pl.ds(f * _HCOLS + j * 16, 16)] = \
                    idx2_v[f, pl.ds(j * 16, 16)]
            return _

        lax.fori_loop(0, _FIELDS, _row, None)

    pl.semaphore_wait(sem_ready, 1)
    g0 = pltpu.async_copy(table_sp.at[idx_a], vals_a, gsem_a)
    g1 = pltpu.async_copy(table_sp.at[idx_b], vals_b, gsem_b)

    for hb, g in ((0, g0), (1, g1)):
        g.wait()
        vb = vals[hb]

        def _row(f, _):
            for j in range(_VPR):
                vals2_v[f, pl.ds(j * 16, 16)] = \
                    vb[pl.ds(f * _HCOLS + j * 16, 16)]
            return _

        lax.fori_loop(0, _FIELDS, _row, None)
        pltpu.sync_copy(vals2_v,
                        out_hbm.at[:, pl.ds(col0 + hb * _HCOLS, _HCOLS)])


_vmem = pltpu.MemorySpace.VMEM @ _vec_mesh

_gather_sc = mpmd.mpmd_map(
    [(_scs_mesh, _scs_fn), (_vec_mesh, _tec_fn)],
    out_types=jax.ShapeDtypeStruct((_FIELDS, _BATCH), jnp.float32),
    scratch_types=[
        _vmem((_FIELDS, _HCOLS), jnp.int32),
        _vmem((_HE,), jnp.int32),
        _vmem((_HE,), jnp.int32),
        _vmem((_HE,), jnp.float32),
        _vmem((_HE,), jnp.float32),
        _vmem((_FIELDS, _HCOLS), jnp.float32),
        pltpu.VMEM_SHARED((_V,), jnp.float32),
        pltpu.SemaphoreType.DMA @ _vec_mesh,
        pltpu.SemaphoreType.DMA @ _vec_mesh,
        pltpu.SemaphoreType.REGULAR @ _vec_mesh,
    ],
)


def kernel(ind, scales):
    if ind.dtype != jnp.int32:
        ind = ind.astype(jnp.int32)
    return _gather_sc(ind.T, scales).T
